# baseline (speedup 1.0000x reference)
import jax
import jax.numpy as jnp
from jax import lax
from jax.experimental import pallas as pl
from jax.experimental.pallas import tpu as pltpu
from jax.experimental.pallas import tpu_sc as plsc

_N = 10000
_E = 320000
_D = 128
_NC = 2
_NS = 16
_NW = _NC * _NS
_C = 128
_NCHUNK = 80
_EPT = _NCHUNK * _C
_EPAD = _NW * _EPT
_NROWS = _N + 16
_RPT = 624
_RPT_LAST = _N - 15 * _RPT
_BS = 1000
_G = _N // _BS

_MESH = plsc.VectorSubcoreMesh(core_axis_name="c", subcore_axis_name="s")


def _write_out(c, s, sh, out):
    @pl.when(s < _NS - 1)
    def _():
        row0 = s * _RPT
        pltpu.sync_copy(sh.at[pl.ds(row0, _RPT)],
                        out.at[pl.ds(c * _N + row0, _RPT)])

    @pl.when(s == _NS - 1)
    def _():
        pltpu.sync_copy(sh.at[pl.ds(15 * _RPT, _RPT_LAST)],
                        out.at[pl.ds(c * _N + 15 * _RPT, _RPT_LAST)])


def _fill_const(buf, val):
    v = jnp.full((16,), val, jnp.float32)

    def row(r, carry):
        for j in range(_D // 16):
            buf[r, pl.ds(j * 16, 16)] = v
        return carry

    lax.fori_loop(0, _C, row, 0)


def _zero_shared(s, zbuf, sh):
    @pl.when(s < _NS - 1)
    def _():
        row0 = s * _RPT
        for i in range(4):
            pltpu.sync_copy(zbuf, sh.at[pl.ds(row0 + i * _C, _C)])
        pltpu.sync_copy(zbuf.at[pl.ds(0, _RPT - 4 * _C)],
                        sh.at[pl.ds(row0 + 4 * _C, _RPT - 4 * _C)])

    @pl.when(s == _NS - 1)
    def _():
        for i in range(5):
            pltpu.sync_copy(zbuf, sh.at[pl.ds(15 * _RPT + i * _C, _C)])


def _load_idx(dst_hbm, base, dst_all, isem):
    def ld(k):
        pltpu.async_copy(dst_hbm.at[pl.ds(base + k * _C, _C)],
                         dst_all.at[k], isem)

    def drain():
        pltpu.make_async_copy(dst_hbm.at[pl.ds(0, _C)], dst_all.at[0],
                              isem).wait()

    _IG = 8
    for j in range(_IG):
        ld(j)

    def group(g, carry):
        b = (g + 1) * _IG
        for j in range(_IG):
            ld(b + j)
        for j in range(_IG):
            drain()
        return carry

    lax.fori_loop(0, _NCHUNK // _IG - 1, group, 0)
    for j in range(_IG):
        drain()


def _sc_agg_body(y_hbm, src_hbm, dst_hbm, acc_out,
                 srcv0, srcv1, dst_all, rows0, rows1, acc_sh,
                 gsem, ssem, isem):
    c = lax.axis_index("c")
    s = lax.axis_index("s")
    gid = c * _NS + s
    base = gid * _EPT
    _load_idx(dst_hbm, base, dst_all, isem)
    _fill_const(rows0, 0.0)
    _zero_shared(s, rows0, acc_sh)
    plsc.subcore_barrier()

    def i_fire(k, sbuf):
        pltpu.async_copy(src_hbm.at[pl.ds(base + k * _C, _C)], sbuf, isem)

    def i_wait():
        pltpu.make_async_copy(src_hbm.at[pl.ds(0, _C)], srcv0, isem).wait()

    def g_fire(sbuf, rbuf):
        pltpu.async_copy(y_hbm.at[sbuf], rbuf, gsem)

    def g_wait(rbuf):
        pltpu.make_async_copy(y_hbm.at[srcv0], rbuf, gsem).wait()

    def s_fire(k, rbuf):
        pltpu.async_copy(rbuf, acc_sh.at[dst_all.at[k]], ssem, add=True)

    def s_wait(rbuf):
        pltpu.make_async_copy(rbuf, acc_sh.at[dst_all.at[0]], ssem).wait()

    pltpu.sync_copy(src_hbm.at[pl.ds(base, _C)], srcv0)
    g_fire(srcv0, rows0)
    pltpu.sync_copy(src_hbm.at[pl.ds(base + _C, _C)], srcv1)
    g_wait(rows0)
    g_fire(srcv1, rows1)
    i_fire(2, srcv0)
    s_fire(0, rows0)

    def pair(t, carry):
        k1 = 2 * t + 1
        g_wait(rows1)
        s_wait(rows0)
        i_wait()
        g_fire(srcv0, rows0)
        i_fire(k1 + 2, srcv1)
        s_fire(k1, rows1)
        g_wait(rows0)
        s_wait(rows1)
        i_wait()
        g_fire(srcv1, rows1)
        i_fire(k1 + 3, srcv0)
        s_fire(k1 + 1, rows0)
        return carry

    lax.fori_loop(0, 38, pair, 0)
    g_wait(rows1)
    s_wait(rows0)
    i_wait()
    g_fire(srcv0, rows0)
    i_fire(79, srcv1)
    s_fire(77, rows1)
    g_wait(rows0)
    s_wait(rows1)
    i_wait()
    g_fire(srcv1, rows1)
    s_fire(78, rows0)
    g_wait(rows1)
    s_wait(rows0)
    s_fire(79, rows1)
    s_wait(rows1)
    plsc.subcore_barrier()
    _write_out(c, s, acc_sh, acc_out)


_sc_agg = pl.kernel(
    _sc_agg_body,
    out_type=jax.ShapeDtypeStruct((_NC * _N, _D), jnp.float32),
    mesh=_MESH,
    scratch_types=[
        pltpu.VMEM((_C,), jnp.int32),
        pltpu.VMEM((_C,), jnp.int32),
        pltpu.VMEM((_NCHUNK, _C), jnp.int32),
        pltpu.VMEM((_C, _D), jnp.float32),
        pltpu.VMEM((_C, _D), jnp.float32),
        pltpu.VMEM_SHARED((_NROWS, _D), jnp.float32),
        pltpu.SemaphoreType.DMA,
        pltpu.SemaphoreType.DMA,
        pltpu.SemaphoreType.DMA,
    ],
)

_DGRP = 5
_NDGRP = _NCHUNK // _DGRP


def _sc_deg_body(dst_hbm, deg_out, dst_all, ones_v, zv, deg_sh, ssem, isem):
    c = lax.axis_index("c")
    s = lax.axis_index("s")
    gid = c * _NS + s
    base = gid * _EPT
    _load_idx(dst_hbm, base, dst_all, isem)
    _fill_const(ones_v, 1.0)
    _fill_const(zv, 0.0)
    _zero_shared(s, zv, deg_sh)
    plsc.subcore_barrier()

    def fire(k):
        pltpu.async_copy(ones_v, deg_sh.at[dst_all.at[k]], ssem, add=True)

    def drain():
        pltpu.make_async_copy(ones_v, deg_sh.at[dst_all.at[0]], ssem).wait()

    for j in range(_DGRP):
        fire(j)

    def group(t, carry):
        b = (t + 1) * _DGRP
        for j in range(_DGRP):
            fire(b + j)
        for j in range(_DGRP):
            drain()
        return carry

    lax.fori_loop(0, _NDGRP - 1, group, 0)
    for j in range(_DGRP):
        drain()
    plsc.subcore_barrier()
    _write_out(c, s, deg_sh, deg_out)


_sc_deg = pl.kernel(
    _sc_deg_body,
    out_type=jax.ShapeDtypeStruct((_NC * _N, _D), jnp.float32),
    mesh=_MESH,
    scratch_types=[
        pltpu.VMEM((_NCHUNK, _C), jnp.int32),
        pltpu.VMEM((_C, _D), jnp.float32),
        pltpu.VMEM((_C, _D), jnp.float32),
        pltpu.VMEM_SHARED((_NROWS, _D), jnp.float32),
        pltpu.SemaphoreType.DMA,
        pltpu.SemaphoreType.DMA,
    ],
)


def _dot_t(a, w):
    return lax.dot_general(a, w, (((1,), (1,)), ((), ())),
                           preferred_element_type=jnp.float32)


def _tc_pre_body(x_ref, wl_ref, wr_ref, y_ref, r_ref):
    xb = x_ref[...]
    y_ref[...] = _dot_t(xb, wl_ref[...])
    r_ref[...] = _dot_t(xb, wr_ref[...])


@jax.jit
def _tc_pre(x, wl, wr):
    return pl.pallas_call(
        _tc_pre_body,
        grid=(_G,),
        in_specs=[
            pl.BlockSpec((_BS, _D), lambda i: (i, 0)),
            pl.BlockSpec((_D, _D), lambda i: (0, 0)),
            pl.BlockSpec((_D, _D), lambda i: (0, 0)),
        ],
        out_specs=[
            pl.BlockSpec((_BS, _D), lambda i: (i, 0)),
            pl.BlockSpec((_BS, _D), lambda i: (i, 0)),
        ],
        out_shape=[
            jax.ShapeDtypeStruct((_N, _D), jnp.float32),
            jax.ShapeDtypeStruct((_N, _D), jnp.float32),
        ],
    )(x, wl, wr)


def _tc_mid_body(aA_ref, aB_ref, dA_ref, dB_ref, r1_ref, b_ref,
                 wl_ref, wr_ref, y2_ref, r2_ref):
    ssum = aA_ref[...] + aB_ref[...]
    deg = dA_ref[:, 0:1] + dB_ref[:, 0:1]
    inv = 1.0 / jnp.maximum(deg, 1.0)
    h = jnp.maximum(ssum * inv + b_ref[...] + r1_ref[...], 0.0)
    y2_ref[...] = _dot_t(h, wl_ref[...])
    r2_ref[...] = _dot_t(h, wr_ref[...])


@jax.jit
def _tc_mid(acc, deg, r1, b, wl, wr):
    return pl.pallas_call(
        _tc_mid_body,
        grid=(_G,),
        in_specs=[
            pl.BlockSpec((_BS, _D), lambda i: (i, 0)),
            pl.BlockSpec((_BS, _D), lambda i: (i + _G, 0)),
            pl.BlockSpec((_BS, _D), lambda i: (i, 0)),
            pl.BlockSpec((_BS, _D), lambda i: (i + _G, 0)),
            pl.BlockSpec((_BS, _D), lambda i: (i, 0)),
            pl.BlockSpec((1, _D), lambda i: (0, 0)),
            pl.BlockSpec((_D, _D), lambda i: (0, 0)),
            pl.BlockSpec((_D, _D), lambda i: (0, 0)),
        ],
        out_specs=[
            pl.BlockSpec((_BS, _D), lambda i: (i, 0)),
            pl.BlockSpec((_BS, _D), lambda i: (i, 0)),
        ],
        out_shape=[
            jax.ShapeDtypeStruct((_N, _D), jnp.float32),
            jax.ShapeDtypeStruct((_N, _D), jnp.float32),
        ],
    )(acc, acc, deg, deg, r1, b, wl, wr)


def _tc_final_body(aA_ref, aB_ref, dA_ref, dB_ref, r2_ref, b_ref, z_ref):
    ssum = aA_ref[...] + aB_ref[...]
    deg = dA_ref[:, 0:1] + dB_ref[:, 0:1]
    inv = 1.0 / jnp.maximum(deg, 1.0)
    z_ref[...] = ssum * inv + b_ref[...] + r2_ref[...]


@jax.jit
def _tc_final(acc, deg, r2, b):
    return pl.pallas_call(
        _tc_final_body,
        grid=(_G,),
        in_specs=[
            pl.BlockSpec((_BS, _D), lambda i: (i, 0)),
            pl.BlockSpec((_BS, _D), lambda i: (i + _G, 0)),
            pl.BlockSpec((_BS, _D), lambda i: (i, 0)),
            pl.BlockSpec((_BS, _D), lambda i: (i + _G, 0)),
            pl.BlockSpec((_BS, _D), lambda i: (i, 0)),
            pl.BlockSpec((1, _D), lambda i: (0, 0)),
        ],
        out_specs=pl.BlockSpec((_BS, _D), lambda i: (i, 0)),
        out_shape=jax.ShapeDtypeStruct((_N, _D), jnp.float32),
    )(acc, acc, deg, deg, r2, b)


def kernel(x, edge_index, W1l, b1l, W1r, W2l, b2l, W2r):
    pad = _EPAD - _E
    src = jnp.concatenate(
        [edge_index[0].astype(jnp.int32), jnp.zeros((pad,), jnp.int32)])
    dst = jnp.concatenate(
        [edge_index[1].astype(jnp.int32), jnp.full((pad,), _N, jnp.int32)])
    b1 = b1l.reshape(1, _D)
    b2 = b2l.reshape(1, _D)

    y1, r1 = _tc_pre(x, W1l, W1r)
    deg = _sc_deg(dst)
    acc1 = _sc_agg(y1, src, dst)
    y2, r2 = _tc_mid(acc1, deg, r1, b1, W2l, W2r)
    acc2 = _sc_agg(y2, src, dst)
    z = _tc_final(acc2, deg, r2, b2)
    return z


# speedup vs baseline: 2.8200x; 2.8200x over previous
import jax
import jax.numpy as jnp
from jax import lax
from jax.experimental import pallas as pl
from jax.experimental.pallas import tpu as pltpu
from jax.experimental.pallas import tpu_sc as plsc

_N = 10000
_E = 320000
_D = 128
_NC = 2
_NS = 16
_NW = _NC * _NS
_C = 128
_NCHUNK = 80
_EPT = _NCHUNK * _C
_EPAD = _NW * _EPT
_NROWS = _N + 16
_RPT = 624
_RPT_LAST = _N - 15 * _RPT
_BS = 1000
_G = _N // _BS

_MESH = plsc.VectorSubcoreMesh(core_axis_name="c", subcore_axis_name="s")


def _write_out(c, s, sh, out):
    @pl.when(s < _NS - 1)
    def _():
        row0 = s * _RPT
        pltpu.sync_copy(sh.at[pl.ds(row0, _RPT)],
                        out.at[pl.ds(c * _N + row0, _RPT)])

    @pl.when(s == _NS - 1)
    def _():
        pltpu.sync_copy(sh.at[pl.ds(15 * _RPT, _RPT_LAST)],
                        out.at[pl.ds(c * _N + 15 * _RPT, _RPT_LAST)])


def _fill_const(buf, val):
    v = jnp.full((16,), val, jnp.float32)

    def row(r, carry):
        for j in range(_D // 16):
            buf[r, pl.ds(j * 16, 16)] = v
        return carry

    lax.fori_loop(0, _C, row, 0)


def _zero_shared(s, zbuf, sh):
    @pl.when(s < _NS - 1)
    def _():
        row0 = s * _RPT
        for i in range(4):
            pltpu.sync_copy(zbuf, sh.at[pl.ds(row0 + i * _C, _C)])
        pltpu.sync_copy(zbuf.at[pl.ds(0, _RPT - 4 * _C)],
                        sh.at[pl.ds(row0 + 4 * _C, _RPT - 4 * _C)])

    @pl.when(s == _NS - 1)
    def _():
        for i in range(5):
            pltpu.sync_copy(zbuf, sh.at[pl.ds(15 * _RPT + i * _C, _C)])


def _load_idx(dst_hbm, base, dst_all, isem):
    def ld(k):
        pltpu.async_copy(dst_hbm.at[pl.ds(base + k * _C, _C)],
                         dst_all.at[k], isem)

    def drain():
        pltpu.make_async_copy(dst_hbm.at[pl.ds(0, _C)], dst_all.at[0],
                              isem).wait()

    _IG = 8
    for j in range(_IG):
        ld(j)

    def group(g, carry):
        b = (g + 1) * _IG
        for j in range(_IG):
            ld(b + j)
        for j in range(_IG):
            drain()
        return carry

    lax.fori_loop(0, _NCHUNK // _IG - 1, group, 0)
    for j in range(_IG):
        drain()


def _sc_agg_body(y_hbm, src_hbm, dst_hbm, acc_out,
                 srcv0, srcv1, dst_all, rows0, rows1, acc_sh,
                 gsem, ssem, isem):
    c = lax.axis_index("c")
    s = lax.axis_index("s")
    gid = c * _NS + s
    base = gid * _EPT
    _load_idx(dst_hbm, base, dst_all, isem)
    _fill_const(rows0, 0.0)
    _zero_shared(s, rows0, acc_sh)
    plsc.subcore_barrier()

    def i_fire(k, sbuf):
        pltpu.async_copy(src_hbm.at[pl.ds(base + k * _C, _C)], sbuf, isem)

    def i_wait():
        pltpu.make_async_copy(src_hbm.at[pl.ds(0, _C)], srcv0, isem).wait()

    def g_fire(sbuf, rbuf):
        pltpu.async_copy(y_hbm.at[sbuf], rbuf, gsem)

    def g_wait(rbuf):
        pltpu.make_async_copy(y_hbm.at[srcv0], rbuf, gsem).wait()

    def s_fire(k, rbuf):
        pltpu.async_copy(rbuf, acc_sh.at[dst_all.at[k]], ssem, add=True)

    def s_wait(rbuf):
        pltpu.make_async_copy(rbuf, acc_sh.at[dst_all.at[0]], ssem).wait()

    pltpu.sync_copy(src_hbm.at[pl.ds(base, _C)], srcv0)
    g_fire(srcv0, rows0)
    pltpu.sync_copy(src_hbm.at[pl.ds(base + _C, _C)], srcv1)
    g_wait(rows0)
    g_fire(srcv1, rows1)
    i_fire(2, srcv0)
    s_fire(0, rows0)

    def pair(t, carry):
        k1 = 2 * t + 1
        g_wait(rows1)
        s_wait(rows0)
        i_wait()
        g_fire(srcv0, rows0)
        i_fire(k1 + 2, srcv1)
        s_fire(k1, rows1)
        g_wait(rows0)
        s_wait(rows1)
        i_wait()
        g_fire(srcv1, rows1)
        i_fire(k1 + 3, srcv0)
        s_fire(k1 + 1, rows0)
        return carry

    lax.fori_loop(0, 38, pair, 0)
    g_wait(rows1)
    s_wait(rows0)
    i_wait()
    g_fire(srcv0, rows0)
    i_fire(79, srcv1)
    s_fire(77, rows1)
    g_wait(rows0)
    s_wait(rows1)
    i_wait()
    g_fire(srcv1, rows1)
    s_fire(78, rows0)
    g_wait(rows1)
    s_wait(rows0)
    s_fire(79, rows1)
    s_wait(rows1)
    plsc.subcore_barrier()
    _write_out(c, s, acc_sh, acc_out)


_sc_agg = pl.kernel(
    _sc_agg_body,
    out_type=jax.ShapeDtypeStruct((_NC * _N, _D), jnp.float32),
    mesh=_MESH,
    scratch_types=[
        pltpu.VMEM((_C,), jnp.int32),
        pltpu.VMEM((_C,), jnp.int32),
        pltpu.VMEM((_NCHUNK, _C), jnp.int32),
        pltpu.VMEM((_C, _D), jnp.float32),
        pltpu.VMEM((_C, _D), jnp.float32),
        pltpu.VMEM_SHARED((_NROWS, _D), jnp.float32),
        pltpu.SemaphoreType.DMA,
        pltpu.SemaphoreType.DMA,
        pltpu.SemaphoreType.DMA,
    ],
)

_DGRP = 5
_NDGRP = _NCHUNK // _DGRP


def _sc_deg_body(dst_hbm, deg_out, dst_all, ones_v, zv, deg_sh, ssem, isem):
    c = lax.axis_index("c")
    s = lax.axis_index("s")
    gid = c * _NS + s
    base = gid * _EPT
    _load_idx(dst_hbm, base, dst_all, isem)
    _fill_const(ones_v, 1.0)
    _fill_const(zv, 0.0)
    _zero_shared(s, zv, deg_sh)
    plsc.subcore_barrier()

    def fire(k):
        pltpu.async_copy(ones_v, deg_sh.at[dst_all.at[k]], ssem, add=True)

    def drain():
        pltpu.make_async_copy(ones_v, deg_sh.at[dst_all.at[0]], ssem).wait()

    for j in range(_DGRP):
        fire(j)

    def group(t, carry):
        b = (t + 1) * _DGRP
        for j in range(_DGRP):
            fire(b + j)
        for j in range(_DGRP):
            drain()
        return carry

    lax.fori_loop(0, _NDGRP - 1, group, 0)
    for j in range(_DGRP):
        drain()
    plsc.subcore_barrier()
    _write_out(c, s, deg_sh, deg_out)


_sc_deg = pl.kernel(
    _sc_deg_body,
    out_type=jax.ShapeDtypeStruct((_NC * _N, _D), jnp.float32),
    mesh=_MESH,
    scratch_types=[
        pltpu.VMEM((_NCHUNK, _C), jnp.int32),
        pltpu.VMEM((_C, _D), jnp.float32),
        pltpu.VMEM((_C, _D), jnp.float32),
        pltpu.VMEM_SHARED((_NROWS, _D), jnp.float32),
        pltpu.SemaphoreType.DMA,
        pltpu.SemaphoreType.DMA,
    ],
)


def _dot_t(a, w):
    return lax.dot_general(a, w, (((1,), (1,)), ((), ())),
                           preferred_element_type=jnp.float32)


def _tc_pre_body(x_ref, wl_ref, wr_ref, y_ref, r_ref):
    xb = x_ref[...]
    y_ref[...] = _dot_t(xb, wl_ref[...])
    r_ref[...] = _dot_t(xb, wr_ref[...])


@jax.jit
def _tc_pre(x, wl, wr):
    return pl.pallas_call(
        _tc_pre_body,
        grid=(_G,),
        in_specs=[
            pl.BlockSpec((_BS, _D), lambda i: (i, 0)),
            pl.BlockSpec((_D, _D), lambda i: (0, 0)),
            pl.BlockSpec((_D, _D), lambda i: (0, 0)),
        ],
        out_specs=[
            pl.BlockSpec((_BS, _D), lambda i: (i, 0)),
            pl.BlockSpec((_BS, _D), lambda i: (i, 0)),
        ],
        out_shape=[
            jax.ShapeDtypeStruct((_N, _D), jnp.float32),
            jax.ShapeDtypeStruct((_N, _D), jnp.float32),
        ],
    )(x, wl, wr)


def _tc_mid_body(aA_ref, aB_ref, dA_ref, dB_ref, r1_ref, b_ref,
                 wl_ref, wr_ref, y2_ref, r2_ref):
    ssum = aA_ref[...] + aB_ref[...]
    deg = dA_ref[:, 0:1] + dB_ref[:, 0:1]
    inv = 1.0 / jnp.maximum(deg, 1.0)
    h = jnp.maximum(ssum * inv + b_ref[...] + r1_ref[...], 0.0)
    y2_ref[...] = _dot_t(h, wl_ref[...])
    r2_ref[...] = _dot_t(h, wr_ref[...])


@jax.jit
def _tc_mid(acc, deg, r1, b, wl, wr):
    return pl.pallas_call(
        _tc_mid_body,
        grid=(_G,),
        in_specs=[
            pl.BlockSpec((_BS, _D), lambda i: (i, 0)),
            pl.BlockSpec((_BS, _D), lambda i: (i + _G, 0)),
            pl.BlockSpec((_BS, _D), lambda i: (i, 0)),
            pl.BlockSpec((_BS, _D), lambda i: (i + _G, 0)),
            pl.BlockSpec((_BS, _D), lambda i: (i, 0)),
            pl.BlockSpec((1, _D), lambda i: (0, 0)),
            pl.BlockSpec((_D, _D), lambda i: (0, 0)),
            pl.BlockSpec((_D, _D), lambda i: (0, 0)),
        ],
        out_specs=[
            pl.BlockSpec((_BS, _D), lambda i: (i, 0)),
            pl.BlockSpec((_BS, _D), lambda i: (i, 0)),
        ],
        out_shape=[
            jax.ShapeDtypeStruct((_N, _D), jnp.float32),
            jax.ShapeDtypeStruct((_N, _D), jnp.float32),
        ],
    )(acc, acc, deg, deg, r1, b, wl, wr)


def _tc_final_body(aA_ref, aB_ref, dA_ref, dB_ref, r2_ref, b_ref, z_ref):
    ssum = aA_ref[...] + aB_ref[...]
    deg = dA_ref[:, 0:1] + dB_ref[:, 0:1]
    inv = 1.0 / jnp.maximum(deg, 1.0)
    z_ref[...] = ssum * inv + b_ref[...] + r2_ref[...]


@jax.jit
def _tc_final(acc, deg, r2, b):
    return pl.pallas_call(
        _tc_final_body,
        grid=(_G,),
        in_specs=[
            pl.BlockSpec((_BS, _D), lambda i: (i, 0)),
            pl.BlockSpec((_BS, _D), lambda i: (i + _G, 0)),
            pl.BlockSpec((_BS, _D), lambda i: (i, 0)),
            pl.BlockSpec((_BS, _D), lambda i: (i + _G, 0)),
            pl.BlockSpec((_BS, _D), lambda i: (i, 0)),
            pl.BlockSpec((1, _D), lambda i: (0, 0)),
        ],
        out_specs=pl.BlockSpec((_BS, _D), lambda i: (i, 0)),
        out_shape=jax.ShapeDtypeStruct((_N, _D), jnp.float32),
    )(acc, acc, deg, deg, r2, b)


def kernel(x, edge_index, W1l, b1l, W1r, W2l, b2l, W2r):
    pad = _EPAD - _E
    src = jnp.concatenate(
        [edge_index[0].astype(jnp.int32),
         (jnp.arange(pad, dtype=jnp.int32) * 16) % _N])
    dst = jnp.concatenate(
        [edge_index[1].astype(jnp.int32), jnp.full((pad,), _N, jnp.int32)])
    b1 = b1l.reshape(1, _D)
    b2 = b2l.reshape(1, _D)

    y1, r1 = _tc_pre(x, W1l, W1r)
    deg = _sc_deg(dst)
    acc1 = _sc_agg(y1, src, dst)
    y2, r2 = _tc_mid(acc1, deg, r1, b1, W2l, W2r)
    acc2 = _sc_agg(y2, src, dst)
    z = _tc_final(acc2, deg, r2, b2)
    return z


# speedup vs baseline: 3.1668x; 1.1230x over previous
import jax
import jax.numpy as jnp
from jax import lax
from jax.experimental import pallas as pl
from jax.experimental.pallas import tpu as pltpu
from jax.experimental.pallas import tpu_sc as plsc

_N = 10000
_E = 320000
_D = 128
_NC = 2
_NS = 16
_NW = _NC * _NS
_C = 128
_NCHUNK = 80
_EPT = _NCHUNK * _C
_EPAD = _NW * _EPT
_NROWS = _N + 16
_RPT = 624
_RPT_LAST = _N - 15 * _RPT
_BS = 1000
_G = _N // _BS

_MESH = plsc.VectorSubcoreMesh(core_axis_name="c", subcore_axis_name="s")


def _write_out(c, s, sh, out):
    @pl.when(s < _NS - 1)
    def _():
        row0 = s * _RPT
        pltpu.sync_copy(sh.at[pl.ds(row0, _RPT)],
                        out.at[pl.ds(c * _N + row0, _RPT)])

    @pl.when(s == _NS - 1)
    def _():
        pltpu.sync_copy(sh.at[pl.ds(15 * _RPT, _RPT_LAST)],
                        out.at[pl.ds(c * _N + 15 * _RPT, _RPT_LAST)])


def _fill_const(buf, val, width=_D):
    v = jnp.full((16,), val, jnp.float32)

    def row(r, carry):
        for j in range(width // 16):
            buf[r, pl.ds(j * 16, 16)] = v
        return carry

    lax.fori_loop(0, _C, row, 0)


def _zero_shared(s, zbuf, sh):
    @pl.when(s < _NS - 1)
    def _():
        row0 = s * _RPT
        for i in range(4):
            pltpu.sync_copy(zbuf, sh.at[pl.ds(row0 + i * _C, _C)])
        pltpu.sync_copy(zbuf.at[pl.ds(0, _RPT - 4 * _C)],
                        sh.at[pl.ds(row0 + 4 * _C, _RPT - 4 * _C)])

    @pl.when(s == _NS - 1)
    def _():
        for i in range(5):
            pltpu.sync_copy(zbuf, sh.at[pl.ds(15 * _RPT + i * _C, _C)])


def _load_idx(dst_hbm, base, dst_all, isem):
    def ld(k):
        pltpu.async_copy(dst_hbm.at[pl.ds(base + k * _C, _C)],
                         dst_all.at[k], isem)

    def drain():
        pltpu.make_async_copy(dst_hbm.at[pl.ds(0, _C)], dst_all.at[0],
                              isem).wait()

    _IG = 8
    for j in range(_IG):
        ld(j)

    def group(g, carry):
        b = (g + 1) * _IG
        for j in range(_IG):
            ld(b + j)
        for j in range(_IG):
            drain()
        return carry

    lax.fori_loop(0, _NCHUNK // _IG - 1, group, 0)
    for j in range(_IG):
        drain()


def _sc_agg_body(y_hbm, src_hbm, dst_hbm, acc_out,
                 srcv0, srcv1, dst_all, rows0, rows1, acc_sh,
                 gsem, ssem, isem):
    c = lax.axis_index("c")
    s = lax.axis_index("s")
    gid = c * _NS + s
    base = gid * _EPT
    _load_idx(dst_hbm, base, dst_all, isem)
    _fill_const(rows0, 0.0)
    _zero_shared(s, rows0, acc_sh)
    plsc.subcore_barrier()

    def i_fire(k, sbuf):
        pltpu.async_copy(src_hbm.at[pl.ds(base + k * _C, _C)], sbuf, isem)

    def i_wait():
        pltpu.make_async_copy(src_hbm.at[pl.ds(0, _C)], srcv0, isem).wait()

    def g_fire(sbuf, rbuf):
        pltpu.async_copy(y_hbm.at[sbuf], rbuf, gsem)

    def g_wait(rbuf):
        pltpu.make_async_copy(y_hbm.at[srcv0], rbuf, gsem).wait()

    def s_fire(k, rbuf):
        pltpu.async_copy(rbuf, acc_sh.at[dst_all.at[k]], ssem, add=True)

    def s_wait(rbuf):
        pltpu.make_async_copy(rbuf, acc_sh.at[dst_all.at[0]], ssem).wait()

    pltpu.sync_copy(src_hbm.at[pl.ds(base, _C)], srcv0)
    g_fire(srcv0, rows0)
    pltpu.sync_copy(src_hbm.at[pl.ds(base + _C, _C)], srcv1)
    g_wait(rows0)
    g_fire(srcv1, rows1)
    i_fire(2, srcv0)
    s_fire(0, rows0)

    def pair(t, carry):
        k1 = 2 * t + 1
        g_wait(rows1)
        s_wait(rows0)
        i_wait()
        g_fire(srcv0, rows0)
        i_fire(k1 + 2, srcv1)
        s_fire(k1, rows1)
        g_wait(rows0)
        s_wait(rows1)
        i_wait()
        g_fire(srcv1, rows1)
        i_fire(k1 + 3, srcv0)
        s_fire(k1 + 1, rows0)
        return carry

    lax.fori_loop(0, 38, pair, 0)
    g_wait(rows1)
    s_wait(rows0)
    i_wait()
    g_fire(srcv0, rows0)
    i_fire(79, srcv1)
    s_fire(77, rows1)
    g_wait(rows0)
    s_wait(rows1)
    i_wait()
    g_fire(srcv1, rows1)
    s_fire(78, rows0)
    g_wait(rows1)
    s_wait(rows0)
    s_fire(79, rows1)
    s_wait(rows1)
    plsc.subcore_barrier()
    _write_out(c, s, acc_sh, acc_out)


_sc_agg = pl.kernel(
    _sc_agg_body,
    out_type=jax.ShapeDtypeStruct((_NC * _N, _D), jnp.float32),
    mesh=_MESH,
    scratch_types=[
        pltpu.VMEM((_C,), jnp.int32),
        pltpu.VMEM((_C,), jnp.int32),
        pltpu.VMEM((_NCHUNK, _C), jnp.int32),
        pltpu.VMEM((_C, _D), jnp.float32),
        pltpu.VMEM((_C, _D), jnp.float32),
        pltpu.VMEM_SHARED((_NROWS, _D), jnp.float32),
        pltpu.SemaphoreType.DMA,
        pltpu.SemaphoreType.DMA,
        pltpu.SemaphoreType.DMA,
    ],
)

_DGRP = 5
_NDGRP = _NCHUNK // _DGRP


def _sc_deg_body(dst_hbm, deg_out, dst_all, ones_v, deg_sh, ssem, isem):
    c = lax.axis_index("c")
    s = lax.axis_index("s")
    gid = c * _NS + s
    base = gid * _EPT
    _load_idx(dst_hbm, base, dst_all, isem)
    _fill_const(ones_v, 0.0, width=16)
    _zero_shared(s, ones_v, deg_sh)
    _fill_const(ones_v, 1.0, width=16)
    plsc.subcore_barrier()

    def fire(k):
        pltpu.async_copy(ones_v, deg_sh.at[dst_all.at[k]], ssem, add=True)

    def drain():
        pltpu.make_async_copy(ones_v, deg_sh.at[dst_all.at[0]], ssem).wait()

    for j in range(_DGRP):
        fire(j)

    def group(t, carry):
        b = (t + 1) * _DGRP
        for j in range(_DGRP):
            fire(b + j)
        for j in range(_DGRP):
            drain()
        return carry

    lax.fori_loop(0, _NDGRP - 1, group, 0)
    for j in range(_DGRP):
        drain()
    plsc.subcore_barrier()
    _write_out(c, s, deg_sh, deg_out)


_sc_deg = pl.kernel(
    _sc_deg_body,
    out_type=jax.ShapeDtypeStruct((_NC * _N, 16), jnp.float32),
    mesh=_MESH,
    scratch_types=[
        pltpu.VMEM((_NCHUNK, _C), jnp.int32),
        pltpu.VMEM((_C, 16), jnp.float32),
        pltpu.VMEM_SHARED((_NROWS, 16), jnp.float32),
        pltpu.SemaphoreType.DMA,
        pltpu.SemaphoreType.DMA,
    ],
)


def _dot_t(a, w):
    return lax.dot_general(a, w, (((1,), (1,)), ((), ())),
                           preferred_element_type=jnp.float32)


def _tc_pre_body(x_ref, wl_ref, wr_ref, y_ref, r_ref):
    xb = x_ref[...]
    y_ref[...] = _dot_t(xb, wl_ref[...])
    r_ref[...] = _dot_t(xb, wr_ref[...])


@jax.jit
def _tc_pre(x, wl, wr):
    return pl.pallas_call(
        _tc_pre_body,
        grid=(_G,),
        in_specs=[
            pl.BlockSpec((_BS, _D), lambda i: (i, 0)),
            pl.BlockSpec((_D, _D), lambda i: (0, 0)),
            pl.BlockSpec((_D, _D), lambda i: (0, 0)),
        ],
        out_specs=[
            pl.BlockSpec((_BS, _D), lambda i: (i, 0)),
            pl.BlockSpec((_BS, _D), lambda i: (i, 0)),
        ],
        out_shape=[
            jax.ShapeDtypeStruct((_N, _D), jnp.float32),
            jax.ShapeDtypeStruct((_N, _D), jnp.float32),
        ],
    )(x, wl, wr)


def _tc_mid_body(aA_ref, aB_ref, dA_ref, dB_ref, r1_ref, b_ref,
                 wl_ref, wr_ref, y2_ref, r2_ref):
    ssum = aA_ref[...] + aB_ref[...]
    deg = dA_ref[:, 0:1] + dB_ref[:, 0:1]
    inv = 1.0 / jnp.maximum(deg, 1.0)
    h = jnp.maximum(ssum * inv + b_ref[...] + r1_ref[...], 0.0)
    y2_ref[...] = _dot_t(h, wl_ref[...])
    r2_ref[...] = _dot_t(h, wr_ref[...])


@jax.jit
def _tc_mid(acc, deg, r1, b, wl, wr):
    return pl.pallas_call(
        _tc_mid_body,
        grid=(_G,),
        in_specs=[
            pl.BlockSpec((_BS, _D), lambda i: (i, 0)),
            pl.BlockSpec((_BS, _D), lambda i: (i + _G, 0)),
            pl.BlockSpec((_BS, 16), lambda i: (i, 0)),
            pl.BlockSpec((_BS, 16), lambda i: (i + _G, 0)),
            pl.BlockSpec((_BS, _D), lambda i: (i, 0)),
            pl.BlockSpec((1, _D), lambda i: (0, 0)),
            pl.BlockSpec((_D, _D), lambda i: (0, 0)),
            pl.BlockSpec((_D, _D), lambda i: (0, 0)),
        ],
        out_specs=[
            pl.BlockSpec((_BS, _D), lambda i: (i, 0)),
            pl.BlockSpec((_BS, _D), lambda i: (i, 0)),
        ],
        out_shape=[
            jax.ShapeDtypeStruct((_N, _D), jnp.float32),
            jax.ShapeDtypeStruct((_N, _D), jnp.float32),
        ],
    )(acc, acc, deg, deg, r1, b, wl, wr)


def _tc_final_body(aA_ref, aB_ref, dA_ref, dB_ref, r2_ref, b_ref, z_ref):
    ssum = aA_ref[...] + aB_ref[...]
    deg = dA_ref[:, 0:1] + dB_ref[:, 0:1]
    inv = 1.0 / jnp.maximum(deg, 1.0)
    z_ref[...] = ssum * inv + b_ref[...] + r2_ref[...]


@jax.jit
def _tc_final(acc, deg, r2, b):
    return pl.pallas_call(
        _tc_final_body,
        grid=(_G,),
        in_specs=[
            pl.BlockSpec((_BS, _D), lambda i: (i, 0)),
            pl.BlockSpec((_BS, _D), lambda i: (i + _G, 0)),
            pl.BlockSpec((_BS, 16), lambda i: (i, 0)),
            pl.BlockSpec((_BS, 16), lambda i: (i + _G, 0)),
            pl.BlockSpec((_BS, _D), lambda i: (i, 0)),
            pl.BlockSpec((1, _D), lambda i: (0, 0)),
        ],
        out_specs=pl.BlockSpec((_BS, _D), lambda i: (i, 0)),
        out_shape=jax.ShapeDtypeStruct((_N, _D), jnp.float32),
    )(acc, acc, deg, deg, r2, b)


def kernel(x, edge_index, W1l, b1l, W1r, W2l, b2l, W2r):
    pad = _EPAD - _E
    src = jnp.concatenate(
        [edge_index[0].astype(jnp.int32),
         (jnp.arange(pad, dtype=jnp.int32) * 16) % _N])
    dst = jnp.concatenate(
        [edge_index[1].astype(jnp.int32), jnp.full((pad,), _N, jnp.int32)])
    b1 = b1l.reshape(1, _D)
    b2 = b2l.reshape(1, _D)

    y1, r1 = _tc_pre(x, W1l, W1r)
    deg = _sc_deg(dst)
    acc1 = _sc_agg(y1, src, dst)
    y2, r2 = _tc_mid(acc1, deg, r1, b1, W2l, W2r)
    acc2 = _sc_agg(y2, src, dst)
    z = _tc_final(acc2, deg, r2, b2)
    return z
